# baseline (speedup 1.0000x reference)
import jax
import jax.numpy as jnp
from jax.experimental import pallas as pl

_GRID = 10
_G2 = _GRID * _GRID
_D = 128
_G2P = 128
_BB = 2048


def _fused(x_ref, wt_ref,
           We1_ref, be1_ref, We2_ref, be2_ref, We3_ref, be3_ref,
           Wd1_ref, bd1_ref, Wd2_ref, bd2_ref, Wd3_ref, bd3_ref,
           Wg1_ref, bg1_ref, Wg2_ref, bg2_ref,
           out_ref):
    eps = 1e-12
    x = x_ref[...]
    wt = wt_ref[...]

    x2 = jnp.sum(x * x, axis=1, keepdims=True)
    w2 = jnp.sum(wt * wt, axis=0, keepdims=True)
    pad = jax.lax.broadcasted_iota(jnp.int32, (1, _G2P), 1) >= _G2
    w2 = jnp.where(pad, jnp.float32(1e30), w2)
    d2 = x2 - 2.0 * jnp.dot(x, wt, preferred_element_type=jnp.float32) + w2
    idx = jnp.argmin(d2, axis=1)
    wi = (idx // _GRID).astype(jnp.float32)[:, None] * 0.1
    wj = (idx % _GRID).astype(jnp.float32)[:, None] * 0.1

    h = jnp.tanh(jnp.dot(x, We1_ref[...], preferred_element_type=jnp.float32) + be1_ref[...])
    h = jnp.tanh(jnp.dot(h, We2_ref[...], preferred_element_type=jnp.float32) + be2_ref[...])
    z_c = jnp.dot(h, We3_ref[...], preferred_element_type=jnp.float32) + be3_ref[...]

    h = jnp.tanh(jnp.dot(z_c, Wd1_ref[...], preferred_element_type=jnp.float32) + bd1_ref[...])
    h = jnp.tanh(jnp.dot(h, Wd2_ref[...], preferred_element_type=jnp.float32) + bd2_ref[...])
    x_hat = jnp.dot(h, Wd3_ref[...], preferred_element_type=jnp.float32) + bd3_ref[...]

    diff = x - x_hat
    d_norm = jnp.sqrt(jnp.sum(diff * diff, axis=1, keepdims=True))
    x_norm = jnp.sqrt(x2)
    xh_norm = jnp.sqrt(jnp.sum(x_hat * x_hat, axis=1, keepdims=True))
    rec_e = d_norm / (x_norm + eps)
    rec_c = jnp.sum(x * x_hat, axis=1, keepdims=True) / (x_norm * xh_norm + eps)

    Wg1 = Wg1_ref[...]
    g = (jnp.dot(z_c, Wg1[0:4, :], preferred_element_type=jnp.float32)
         + rec_e * Wg1[4:5, :] + rec_c * Wg1[5:6, :]
         + wi * Wg1[6:7, :] + wj * Wg1[7:8, :]
         + bg1_ref[...])
    g = jnp.tanh(g)
    logits = jnp.dot(g, Wg2_ref[...], preferred_element_type=jnp.float32) + bg2_ref[...]
    m = jnp.max(logits, axis=1, keepdims=True)
    e = jnp.exp(logits - m)
    out_ref[...] = e / jnp.sum(e, axis=1, keepdims=True)


def kernel(input, som_weights, We1, be1, We2, be2, We3, be3,
           Wd1, bd1, Wd2, bd2, Wd3, bd3, Wg1, bg1, Wg2, bg2):
    B = input.shape[0]
    flat_t = som_weights.reshape(_G2, _D).T
    wt = jnp.pad(flat_t, ((0, 0), (0, _G2P - _G2)))

    def row_spec(n):
        return pl.BlockSpec((_BB, n), lambda i: (i, 0))

    def full_spec(a):
        nd = a.ndim
        return pl.BlockSpec(a.shape, lambda i: (0,) * nd)

    weights = (wt,
               We1, be1.reshape(1, -1), We2, be2.reshape(1, -1),
               We3, be3.reshape(1, -1),
               Wd1, bd1.reshape(1, -1), Wd2, bd2.reshape(1, -1),
               Wd3, bd3.reshape(1, -1),
               Wg1, bg1.reshape(1, -1), Wg2, bg2.reshape(1, -1))

    gamma = pl.pallas_call(
        _fused,
        grid=(B // _BB,),
        in_specs=[row_spec(_D)] + [full_spec(w) for w in weights],
        out_specs=row_spec(4),
        out_shape=jax.ShapeDtypeStruct((B, 4), jnp.float32),
    )(input, *weights)
    return gamma


# speedup vs baseline: 1.3870x; 1.3870x over previous
import jax
import jax.numpy as jnp
from jax.experimental import pallas as pl

_GRID = 10
_G2 = _GRID * _GRID
_D = 128
_BB = 2048


def _nt(a, b):
    return jax.lax.dot_general(a, b, (((1,), (1,)), ((), ())),
                               preferred_element_type=jnp.float32)


def _tt(w, act):
    return jax.lax.dot_general(w, act, (((0,), (0,)), ((), ())),
                               preferred_element_type=jnp.float32)


def _tn(w, rows):
    return jax.lax.dot_general(w, rows, (((0,), (1,)), ((), ())),
                               preferred_element_type=jnp.float32)


def _fused(x_ref, flat_ref,
           We1_ref, be1_ref, We2_ref, be2_ref, We3_ref, be3_ref,
           Wd1_ref, bd1_ref, Wd2_ref, bd2_ref, Wd3_ref, bd3_ref,
           Wg1_ref, bg1_ref, Wg2_ref, bg2_ref,
           out_ref):
    eps = 1e-12
    x = x_ref[...]
    flat = flat_ref[...]

    w2 = jnp.sum(flat * flat, axis=1, keepdims=True)
    s = w2 - 2.0 * _nt(flat, x)
    smin = jnp.min(s, axis=0, keepdims=True)
    row = jax.lax.broadcasted_iota(jnp.int32, (_G2, 1), 0)
    idx = jnp.min(jnp.where(s <= smin, row, _G2), axis=0, keepdims=True)
    wi = (idx // _GRID).astype(jnp.float32) * 0.1
    wj = (idx % _GRID).astype(jnp.float32) * 0.1

    ones_row = jnp.ones((1, _D), dtype=jnp.float32)
    x2 = _nt(ones_row, x * x)
    x_norm = jnp.sqrt(x2)

    h = jnp.tanh(_tn(We1_ref[...], x) + be1_ref[...].T)
    h = jnp.tanh(_tt(We2_ref[...], h) + be2_ref[...].T)
    z_c = _tt(We3_ref[...], h) + be3_ref[...].T

    h = jnp.tanh(_tt(Wd1_ref[...], z_c) + bd1_ref[...].T)
    h = jnp.tanh(_tt(Wd2_ref[...], h) + bd2_ref[...].T)
    x_hat = _tt(Wd3_ref[...], h) + bd3_ref[...].T

    C = _nt(Wd3_ref[...], x)
    xxh = jnp.sum(h * C, axis=0, keepdims=True) + _nt(bd3_ref[...], x)
    xh2 = jnp.sum(x_hat * x_hat, axis=0, keepdims=True)
    diff2 = jnp.maximum(x2 - 2.0 * xxh + xh2, 0.0)
    rec_e = jnp.sqrt(diff2) / (x_norm + eps)
    rec_c = xxh / (x_norm * jnp.sqrt(xh2) + eps)

    z = jnp.concatenate([z_c, rec_e, rec_c, wi, wj], axis=0)
    g = jnp.tanh(_tt(Wg1_ref[...], z) + bg1_ref[...].T)
    logits = _tt(Wg2_ref[...], g) + bg2_ref[...].T
    m = jnp.max(logits, axis=0, keepdims=True)
    e = jnp.exp(logits - m)
    gamma = e / jnp.sum(e, axis=0, keepdims=True)
    out_ref[...] = gamma.T


def kernel(input, som_weights, We1, be1, We2, be2, We3, be3,
           Wd1, bd1, Wd2, bd2, Wd3, bd3, Wg1, bg1, Wg2, bg2):
    B = input.shape[0]
    flat = som_weights.reshape(_G2, _D)

    def full_spec(a):
        nd = a.ndim
        return pl.BlockSpec(a.shape, lambda i: (0,) * nd)

    weights = (flat,
               We1, be1.reshape(1, -1), We2, be2.reshape(1, -1),
               We3, be3.reshape(1, -1),
               Wd1, bd1.reshape(1, -1), Wd2, bd2.reshape(1, -1),
               Wd3, bd3.reshape(1, -1),
               Wg1, bg1.reshape(1, -1), Wg2, bg2.reshape(1, -1))

    gamma = pl.pallas_call(
        _fused,
        grid=(B // _BB,),
        in_specs=[pl.BlockSpec((_BB, _D), lambda i: (i, 0))]
                 + [full_spec(w) for w in weights],
        out_specs=pl.BlockSpec((_BB, 4), lambda i: (i, 0)),
        out_shape=jax.ShapeDtypeStruct((B, 4), jnp.float32),
    )(input, *weights)
    return gamma


# speedup vs baseline: 1.5708x; 1.1325x over previous
import jax
import jax.numpy as jnp
from jax.experimental import pallas as pl

_GRID = 10
_G2 = _GRID * _GRID
_D = 128
_BB = 4096


def _nt(a, b):
    return jax.lax.dot_general(a, b, (((1,), (1,)), ((), ())),
                               preferred_element_type=jnp.float32)


def _tt(w, act):
    return jax.lax.dot_general(w, act, (((0,), (0,)), ((), ())),
                               preferred_element_type=jnp.float32)


def _tn(w, rows):
    return jax.lax.dot_general(w, rows, (((0,), (1,)), ((), ())),
                               preferred_element_type=jnp.float32)


def _fused(x_ref, flat_ref,
           We1_ref, be1_ref, We2_ref, be2_ref, We3_ref, be3_ref,
           Wd1_ref, bd1_ref, Wd2_ref, bd2_ref, Wd3_ref, bd3_ref,
           Wg1_ref, bg1_ref, Wg2_ref, bg2_ref,
           out_ref):
    eps = 1e-12
    x = x_ref[...]
    flat = flat_ref[...]

    w2 = jnp.sum(flat * flat, axis=1, keepdims=True)
    s = w2 - 2.0 * _nt(flat, x)
    smin = jnp.min(s, axis=0, keepdims=True)
    row = jax.lax.broadcasted_iota(jnp.int32, (_G2, 1), 0)
    idx = jnp.min(jnp.where(s <= smin, row, _G2), axis=0, keepdims=True)
    wi = (idx // _GRID).astype(jnp.float32) * 0.1
    wj = (idx % _GRID).astype(jnp.float32) * 0.1

    ones_row = jnp.ones((1, _D), dtype=jnp.float32)
    x2 = _nt(ones_row, x * x)
    x_norm = jnp.sqrt(x2)

    h = jnp.tanh(_tn(We1_ref[...], x) + be1_ref[...].T)
    h = jnp.tanh(_tt(We2_ref[...], h) + be2_ref[...].T)
    z_c = _tt(We3_ref[...], h) + be3_ref[...].T

    h = jnp.tanh(_tt(Wd1_ref[...], z_c) + bd1_ref[...].T)
    h = jnp.tanh(_tt(Wd2_ref[...], h) + bd2_ref[...].T)
    x_hat = _tt(Wd3_ref[...], h) + bd3_ref[...].T

    C = _nt(Wd3_ref[...], x)
    xxh = jnp.sum(h * C, axis=0, keepdims=True) + _nt(bd3_ref[...], x)
    xh2 = jnp.sum(x_hat * x_hat, axis=0, keepdims=True)
    diff2 = jnp.maximum(x2 - 2.0 * xxh + xh2, 0.0)
    rec_e = jnp.sqrt(diff2) / (x_norm + eps)
    rec_c = xxh / (x_norm * jnp.sqrt(xh2) + eps)

    z = jnp.concatenate([z_c, rec_e, rec_c, wi, wj], axis=0)
    g = jnp.tanh(_tt(Wg1_ref[...], z) + bg1_ref[...].T)
    logits = _tt(Wg2_ref[...], g) + bg2_ref[...].T
    m = jnp.max(logits, axis=0, keepdims=True)
    e = jnp.exp(logits - m)
    gamma = e / jnp.sum(e, axis=0, keepdims=True)
    out_ref[...] = gamma.T


def kernel(input, som_weights, We1, be1, We2, be2, We3, be3,
           Wd1, bd1, Wd2, bd2, Wd3, bd3, Wg1, bg1, Wg2, bg2):
    B = input.shape[0]
    flat = som_weights.reshape(_G2, _D)

    def full_spec(a):
        nd = a.ndim
        return pl.BlockSpec(a.shape, lambda i: (0,) * nd)

    weights = (flat,
               We1, be1.reshape(1, -1), We2, be2.reshape(1, -1),
               We3, be3.reshape(1, -1),
               Wd1, bd1.reshape(1, -1), Wd2, bd2.reshape(1, -1),
               Wd3, bd3.reshape(1, -1),
               Wg1, bg1.reshape(1, -1), Wg2, bg2.reshape(1, -1))

    gamma = pl.pallas_call(
        _fused,
        grid=(B // _BB,),
        in_specs=[pl.BlockSpec((_BB, _D), lambda i: (i, 0))]
                 + [full_spec(w) for w in weights],
        out_specs=pl.BlockSpec((_BB, 4), lambda i: (i, 0)),
        out_shape=jax.ShapeDtypeStruct((B, 4), jnp.float32),
    )(input, *weights)
    return gamma


# speedup vs baseline: 1.5819x; 1.0071x over previous
import jax
import jax.numpy as jnp
from jax.experimental import pallas as pl

_GRID = 10
_G2 = _GRID * _GRID
_D = 128
_BB = 8192


def _nt(a, b):
    return jax.lax.dot_general(a, b, (((1,), (1,)), ((), ())),
                               preferred_element_type=jnp.float32)


def _tt(w, act):
    return jax.lax.dot_general(w, act, (((0,), (0,)), ((), ())),
                               preferred_element_type=jnp.float32)


def _tn(w, rows):
    return jax.lax.dot_general(w, rows, (((0,), (1,)), ((), ())),
                               preferred_element_type=jnp.float32)


def _fused(x_ref, flat_ref,
           We1_ref, be1_ref, We2_ref, be2_ref, We3_ref, be3_ref,
           Wd1_ref, bd1_ref, Wd2_ref, bd2_ref, Wd3_ref, bd3_ref,
           Wg1_ref, bg1_ref, Wg2_ref, bg2_ref,
           out_ref):
    eps = 1e-12
    x = x_ref[...]
    flat = flat_ref[...]

    w2 = jnp.sum(flat * flat, axis=1, keepdims=True)
    s = w2 - 2.0 * _nt(flat, x)
    smin = jnp.min(s, axis=0, keepdims=True)
    row = jax.lax.broadcasted_iota(jnp.int32, (_G2, 1), 0)
    idx = jnp.min(jnp.where(s <= smin, row, _G2), axis=0, keepdims=True)
    wi = (idx // _GRID).astype(jnp.float32) * 0.1
    wj = (idx % _GRID).astype(jnp.float32) * 0.1

    ones_row = jnp.ones((1, _D), dtype=jnp.float32)
    x2 = _nt(ones_row, x * x)
    x_norm = jnp.sqrt(x2)

    h = jnp.tanh(_tn(We1_ref[...], x) + be1_ref[...].T)
    h = jnp.tanh(_tt(We2_ref[...], h) + be2_ref[...].T)
    z_c = _tt(We3_ref[...], h) + be3_ref[...].T

    h = jnp.tanh(_tt(Wd1_ref[...], z_c) + bd1_ref[...].T)
    h = jnp.tanh(_tt(Wd2_ref[...], h) + bd2_ref[...].T)
    x_hat = _tt(Wd3_ref[...], h) + bd3_ref[...].T

    C = _nt(Wd3_ref[...], x)
    xxh = jnp.sum(h * C, axis=0, keepdims=True) + _nt(bd3_ref[...], x)
    xh2 = jnp.sum(x_hat * x_hat, axis=0, keepdims=True)
    diff2 = jnp.maximum(x2 - 2.0 * xxh + xh2, 0.0)
    rec_e = jnp.sqrt(diff2) / (x_norm + eps)
    rec_c = xxh / (x_norm * jnp.sqrt(xh2) + eps)

    z = jnp.concatenate([z_c, rec_e, rec_c, wi, wj], axis=0)
    g = jnp.tanh(_tt(Wg1_ref[...], z) + bg1_ref[...].T)
    logits = _tt(Wg2_ref[...], g) + bg2_ref[...].T
    m = jnp.max(logits, axis=0, keepdims=True)
    e = jnp.exp(logits - m)
    gamma = e / jnp.sum(e, axis=0, keepdims=True)
    out_ref[...] = gamma.T


def kernel(input, som_weights, We1, be1, We2, be2, We3, be3,
           Wd1, bd1, Wd2, bd2, Wd3, bd3, Wg1, bg1, Wg2, bg2):
    B = input.shape[0]
    flat = som_weights.reshape(_G2, _D)

    def full_spec(a):
        nd = a.ndim
        return pl.BlockSpec(a.shape, lambda i: (0,) * nd)

    weights = (flat,
               We1, be1.reshape(1, -1), We2, be2.reshape(1, -1),
               We3, be3.reshape(1, -1),
               Wd1, bd1.reshape(1, -1), Wd2, bd2.reshape(1, -1),
               Wd3, bd3.reshape(1, -1),
               Wg1, bg1.reshape(1, -1), Wg2, bg2.reshape(1, -1))

    gamma = pl.pallas_call(
        _fused,
        grid=(B // _BB,),
        in_specs=[pl.BlockSpec((_BB, _D), lambda i: (i, 0))]
                 + [full_spec(w) for w in weights],
        out_specs=pl.BlockSpec((_BB, 4), lambda i: (i, 0)),
        out_shape=jax.ShapeDtypeStruct((B, 4), jnp.float32),
    )(input, *weights)
    return gamma


# speedup vs baseline: 1.6730x; 1.0576x over previous
import jax
import jax.numpy as jnp
from jax.experimental import pallas as pl

_GRID = 10
_G2 = _GRID * _GRID
_D = 128
_BB = 8192


def _nt(a, b):
    return jax.lax.dot_general(a, b, (((1,), (1,)), ((), ())),
                               preferred_element_type=jnp.float32)


def _tt(w, act):
    return jax.lax.dot_general(w, act, (((0,), (0,)), ((), ())),
                               preferred_element_type=jnp.float32)


def _tt_bf(w, act):
    return jax.lax.dot_general(w.astype(jnp.bfloat16), act.astype(jnp.bfloat16),
                               (((0,), (0,)), ((), ())),
                               preferred_element_type=jnp.float32)


def _fused(x_ref, flat_ref,
           We1_ref, be1_ref, We2_ref, be2_ref, We3_ref, be3_ref,
           Wd1_ref, bd1_ref, Wd2_ref, bd2_ref, Wd3_ref, bd3_ref,
           Wg1_ref, bg1_ref, Wg2_ref, bg2_ref,
           out_ref):
    eps = 1e-12
    x = x_ref[...]
    flat = flat_ref[...]

    w2 = jnp.sum(flat * flat, axis=1, keepdims=True)
    s = w2 - 2.0 * _nt(flat, x)
    smin = jnp.min(s, axis=0, keepdims=True)
    row = jax.lax.broadcasted_iota(jnp.int32, (_G2, 1), 0)
    idx = jnp.min(jnp.where(s <= smin, row, _G2), axis=0, keepdims=True)
    wi = (idx // _GRID).astype(jnp.float32) * 0.1
    wj = (idx % _GRID).astype(jnp.float32) * 0.1

    ones_row = jnp.ones((1, _D), dtype=jnp.float32)
    x2 = _nt(ones_row, x * x)
    x_norm = jnp.sqrt(x2)

    A = jnp.concatenate([We1_ref[...].T, Wd3_ref[...]], axis=0)
    P = jax.lax.dot_general(A.astype(jnp.bfloat16),
                            x.astype(jnp.bfloat16),
                            (((1,), (1,)), ((), ())),
                            preferred_element_type=jnp.float32)
    h = jnp.tanh(P[0:64] + be1_ref[...].T)
    C = P[64:128]

    h = jnp.tanh(_tt_bf(We2_ref[...], h) + be2_ref[...].T)
    z_c = _tt_bf(We3_ref[...], h) + be3_ref[...].T
    h = jnp.tanh(_tt_bf(Wd1_ref[...], z_c) + bd1_ref[...].T)
    h = jnp.tanh(_tt_bf(Wd2_ref[...], h) + bd2_ref[...].T)
    x_hat = _tt_bf(Wd3_ref[...], h) + bd3_ref[...].T

    xxh = jnp.sum(h * C, axis=0, keepdims=True) + _nt(bd3_ref[...], x)
    xh2 = jnp.sum(x_hat * x_hat, axis=0, keepdims=True)
    diff2 = jnp.maximum(x2 - 2.0 * xxh + xh2, 0.0)
    rec_e = jnp.sqrt(diff2) / (x_norm + eps)
    rec_c = xxh / (x_norm * jnp.sqrt(xh2) + eps)

    z = jnp.concatenate([z_c, rec_e, rec_c, wi, wj], axis=0)
    g = jnp.tanh(_tt(Wg1_ref[...], z) + bg1_ref[...].T)
    logits = _tt(Wg2_ref[...], g) + bg2_ref[...].T
    m = jnp.max(logits, axis=0, keepdims=True)
    e = jnp.exp(logits - m)
    gamma = e / jnp.sum(e, axis=0, keepdims=True)
    out_ref[...] = gamma.T


def kernel(input, som_weights, We1, be1, We2, be2, We3, be3,
           Wd1, bd1, Wd2, bd2, Wd3, bd3, Wg1, bg1, Wg2, bg2):
    B = input.shape[0]
    flat = som_weights.reshape(_G2, _D)

    def full_spec(a):
        nd = a.ndim
        return pl.BlockSpec(a.shape, lambda i: (0,) * nd)

    weights = (flat,
               We1, be1.reshape(1, -1), We2, be2.reshape(1, -1),
               We3, be3.reshape(1, -1),
               Wd1, bd1.reshape(1, -1), Wd2, bd2.reshape(1, -1),
               Wd3, bd3.reshape(1, -1),
               Wg1, bg1.reshape(1, -1), Wg2, bg2.reshape(1, -1))

    gamma = pl.pallas_call(
        _fused,
        grid=(B // _BB,),
        in_specs=[pl.BlockSpec((_BB, _D), lambda i: (i, 0))]
                 + [full_spec(w) for w in weights],
        out_specs=pl.BlockSpec((_BB, 4), lambda i: (i, 0)),
        out_shape=jax.ShapeDtypeStruct((B, 4), jnp.float32),
    )(input, *weights)
    return gamma
